# baseline (speedup 1.0000x reference)
import dataclasses
import functools

import jax
import jax.numpy as jnp
from jax import lax
from jax.experimental import pallas as pl
from jax.experimental.pallas import tpu as pltpu
from jax.experimental.pallas import tpu_sc as plsc

N = 10000
E = 160000
DIN = 256
DH = 512
DOUT = 256
HALF = 128

NC = 2
NS = 16
B = 80
NB = E // NS // B
CH = 25
NCH = NB // CH
EPS = E // NS
NPAD = 10240
RPS = NPAD // NS
ZR = 128
MB = 400
G = N // MB

@functools.lru_cache(maxsize=None)
def _sc_params():
    cp = pltpu.CompilerParams()
    if "needs_layout_passes" in pltpu.CompilerParams.__dataclass_fields__:
        cp = dataclasses.replace(cp, needs_layout_passes=False)
    return cp


@functools.lru_cache(maxsize=None)
def _mesh():
    return plsc.VectorSubcoreMesh(
        core_axis_name="c", subcore_axis_name="s", num_cores=NC, num_subcores=NS
    )


def _sc_hist(row1d, col1d):
    @functools.partial(
        pl.kernel,
        out_type=(
            jax.ShapeDtypeStruct((NPAD,), jnp.float32),
            jax.ShapeDtypeStruct((NPAD,), jnp.float32),
        ),
        mesh=_mesh(),
        scratch_types=[
            pltpu.VMEM_SHARED((NS, NPAD), jnp.float32),
            pltpu.VMEM((EPS,), jnp.int32),
            pltpu.VMEM((NPAD,), jnp.float32),
            pltpu.VMEM((NS, RPS), jnp.float32),
            pltpu.VMEM((RPS,), jnp.float32),
        ],
        compiler_params=_sc_params(),
    )
    def k(row_hbm, col_hbm, cr_hbm, cc_hbm, shared, idxv, local, mbuf, res):
        cid = lax.axis_index("c")
        sid = lax.axis_index("s")

        @pl.loop(0, NPAD, step=16)
        def _(i):
            local[pl.ds(i, 16)] = jnp.zeros((16,), jnp.float32)

        def run(idx_hbm, out_hbm):
            pltpu.sync_copy(idx_hbm.at[sid], idxv)
            ones16 = jnp.ones((16,), jnp.float32)

            @pl.loop(0, EPS, step=16)
            def _(e):
                plsc.addupdate_scatter(local, [idxv[pl.ds(e, 16)]], ones16)

            pltpu.sync_copy(local, shared.at[sid])
            plsc.subcore_barrier()
            for r in range(NS):
                pltpu.sync_copy(shared.at[r, pl.ds(sid * RPS, RPS)],
                                mbuf.at[r])

            @pl.loop(0, RPS, step=16)
            def _(j):
                acc = mbuf[0, pl.ds(j, 16)]
                for r in range(1, NS):
                    acc = acc + mbuf[r, pl.ds(j, 16)]
                res[pl.ds(j, 16)] = acc

            pltpu.sync_copy(res, out_hbm.at[pl.ds(sid * RPS, RPS)])

        @pl.when(cid == 0)
        def _():
            run(row_hbm, cr_hbm)

        @pl.when(cid == 1)
        def _():
            run(col_hbm, cc_hbm)

    return k(row1d, col1d)


def _sc_propagate(ya, yb, row3d, col3d):
    @functools.partial(
        pl.kernel,
        out_type=(
            jax.ShapeDtypeStruct((NPAD, HALF), jnp.float32),
            jax.ShapeDtypeStruct((NPAD, HALF), jnp.float32),
        ),
        mesh=_mesh(),
        scratch_types=[
            pltpu.VMEM_SHARED((NPAD, HALF), jnp.float32),
            pltpu.VMEM((CH, B), jnp.int32),
            pltpu.VMEM((CH, B), jnp.int32),
            pltpu.VMEM((B, HALF), jnp.float32),
            pltpu.VMEM((B, HALF), jnp.float32),
            pltpu.SemaphoreType.DMA,
            pltpu.SemaphoreType.DMA,
        ],
    )
    def k(ya_hbm, yb_hbm, row_hbm, col_hbm, za_hbm, zb_hbm,
          shared, rowc, colc, buf0, buf1, sem0, sem1):
        cid = lax.axis_index("c")
        sid = lax.axis_index("s")

        @pl.loop(0, B)
        def _(i):
            @pl.loop(0, HALF, step=16)
            def _(j):
                buf0[i, pl.ds(j, 16)] = jnp.zeros((16,), jnp.float32)

        @pl.loop(0, RPS // B)
        def _(i):
            pltpu.sync_copy(buf0, shared.at[pl.ds(sid * RPS + i * B, B)])

        plsc.subcore_barrier()

        def run(y_hbm, z_hbm):
            @pl.loop(0, NCH)
            def _(g):
                blk = sid * NCH + g
                pltpu.sync_copy(row_hbm.at[blk], rowc)
                pltpu.sync_copy(col_hbm.at[blk], colc)
                pltpu.async_copy(y_hbm.at[rowc.at[0]], buf0, sem0)

                @pl.loop(0, CH - 1, step=2)
                def _(b):
                    pltpu.async_copy(y_hbm.at[rowc.at[b + 1]], buf1, sem1)
                    pltpu.make_async_copy(y_hbm.at[rowc.at[b]], buf0,
                                          sem0).wait()
                    pltpu.sync_copy(buf0, shared.at[colc.at[b]], add=True)
                    pltpu.async_copy(y_hbm.at[rowc.at[b + 2]], buf0, sem0)
                    pltpu.make_async_copy(y_hbm.at[rowc.at[b + 1]], buf1,
                                          sem1).wait()
                    pltpu.sync_copy(buf1, shared.at[colc.at[b + 1]], add=True)

                pltpu.make_async_copy(y_hbm.at[rowc.at[CH - 1]], buf0,
                                      sem0).wait()
                pltpu.sync_copy(buf0, shared.at[colc.at[CH - 1]], add=True)

            plsc.subcore_barrier()

            @pl.loop(0, RPS // ZR)
            def _(i):
                sl = pl.ds(sid * RPS + i * ZR, ZR)
                pltpu.sync_copy(shared.at[sl], z_hbm.at[sl])

        @pl.when(cid == 0)
        def _():
            run(ya_hbm, za_hbm)

        @pl.when(cid == 1)
        def _():
            run(yb_hbm, zb_hbm)

    return k(ya, yb, row3d, col3d)


def _tc_prep(cnt_r, cnt_c, x):
    def body(cr, cc, xr, dis_o, dis2_o, ya_o, yb_o):
        dis = lax.rsqrt(cr[...] + 1.0)
        dis2 = lax.rsqrt(cc[...] + 1.0)
        dis_o[...] = dis
        dis2_o[...] = dis2
        ya_o[...] = dis * xr[:, :HALF]
        yb_o[...] = dis * xr[:, HALF:]

    return pl.pallas_call(
        body,
        grid=(G,),
        in_specs=[
            pl.BlockSpec((MB, 1), lambda i: (i, 0)),
            pl.BlockSpec((MB, 1), lambda i: (i, 0)),
            pl.BlockSpec((MB, DIN), lambda i: (i, 0)),
        ],
        out_specs=[
            pl.BlockSpec((MB, 1), lambda i: (i, 0)),
            pl.BlockSpec((MB, 1), lambda i: (i, 0)),
            pl.BlockSpec((MB, HALF), lambda i: (i, 0)),
            pl.BlockSpec((MB, HALF), lambda i: (i, 0)),
        ],
        out_shape=(
            jax.ShapeDtypeStruct((N, 1), jnp.float32),
            jax.ShapeDtypeStruct((N, 1), jnp.float32),
            jax.ShapeDtypeStruct((N, HALF), jnp.float32),
            jax.ShapeDtypeStruct((N, HALF), jnp.float32),
        ),
    )(cnt_r, cnt_c, x)


def _tc_mid(z1a, z1b, y1a, y1b, dis, dis2, W1, b1r, W2):
    def body(za, zb, ya, yb, d1, d2, w1, b1, w2, oa, ob):
        t = jnp.concatenate([za[...] + ya[...], zb[...] + yb[...]], axis=1)
        t = (t * d1[...]).astype(jnp.bfloat16)
        h = jnp.dot(t, w1[...].astype(jnp.bfloat16),
                    preferred_element_type=jnp.float32) + b1[...]
        h = jnp.maximum(h, 0.0).astype(jnp.bfloat16)
        y2 = jnp.dot(h, w2[...].astype(jnp.bfloat16),
                     preferred_element_type=jnp.float32) * d2[...]
        oa[...] = y2[:, :HALF]
        ob[...] = y2[:, HALF:]

    return pl.pallas_call(
        body,
        grid=(G,),
        in_specs=[
            pl.BlockSpec((MB, HALF), lambda i: (i, 0)),
            pl.BlockSpec((MB, HALF), lambda i: (i, 0)),
            pl.BlockSpec((MB, HALF), lambda i: (i, 0)),
            pl.BlockSpec((MB, HALF), lambda i: (i, 0)),
            pl.BlockSpec((MB, 1), lambda i: (i, 0)),
            pl.BlockSpec((MB, 1), lambda i: (i, 0)),
            pl.BlockSpec((DIN, DH), lambda i: (0, 0)),
            pl.BlockSpec((1, DH), lambda i: (0, 0)),
            pl.BlockSpec((DH, DOUT), lambda i: (0, 0)),
        ],
        out_specs=[
            pl.BlockSpec((MB, HALF), lambda i: (i, 0)),
            pl.BlockSpec((MB, HALF), lambda i: (i, 0)),
        ],
        out_shape=(
            jax.ShapeDtypeStruct((N, HALF), jnp.float32),
            jax.ShapeDtypeStruct((N, HALF), jnp.float32),
        ),
    )(z1a, z1b, y1a, y1b, dis, dis2, W1, b1r, W2)


def _tc_final(z2a, z2b, y2a, y2b, dis2, b2r):
    def body(za, zb, ya, yb, d2, b2, o):
        t = jnp.concatenate([za[...] + ya[...], zb[...] + yb[...]], axis=1)
        o[...] = t * d2[...] + b2[...]

    return pl.pallas_call(
        body,
        grid=(G,),
        in_specs=[
            pl.BlockSpec((MB, HALF), lambda i: (i, 0)),
            pl.BlockSpec((MB, HALF), lambda i: (i, 0)),
            pl.BlockSpec((MB, HALF), lambda i: (i, 0)),
            pl.BlockSpec((MB, HALF), lambda i: (i, 0)),
            pl.BlockSpec((MB, 1), lambda i: (i, 0)),
            pl.BlockSpec((1, DOUT), lambda i: (0, 0)),
        ],
        out_specs=pl.BlockSpec((MB, DOUT), lambda i: (i, 0)),
        out_shape=jax.ShapeDtypeStruct((N, DOUT), jnp.float32),
    )(z2a, z2b, y2a, y2b, dis2, b2r)


def kernel(x, edge_index, W1, b1, W2, b2):
    row3d = edge_index[0].reshape(NS * NCH, CH, B)
    col3d = edge_index[1].reshape(NS * NCH, CH, B)
    cnt_r, cnt_c = _sc_hist(edge_index[0].reshape(NS, EPS),
                            edge_index[1].reshape(NS, EPS))
    dis, dis2, y1a, y1b = _tc_prep(cnt_r.reshape(NPAD, 1),
                                   cnt_c.reshape(NPAD, 1), x)
    z1a, z1b = _sc_propagate(y1a, y1b, row3d, col3d)
    y2a, y2b = _tc_mid(z1a, z1b, y1a, y1b, dis, dis2, W1,
                       b1.reshape(1, DH), W2)
    z2a, z2b = _sc_propagate(y2a, y2b, row3d, col3d)
    return _tc_final(z2a, z2b, y2a, y2b, dis2, b2.reshape(1, DOUT))


# speedup vs baseline: 1.1195x; 1.1195x over previous
import dataclasses
import functools

import jax
import jax.numpy as jnp
from jax import lax
from jax.experimental import pallas as pl
from jax.experimental.pallas import tpu as pltpu
from jax.experimental.pallas import tpu_sc as plsc

N = 10000
E = 160000
DIN = 256
DH = 512
DOUT = 256
HALF = 128

NC = 2
NS = 16
B = 80
NB = E // NS // B
CH = 25
NCH = NB // CH
EPS = E // NS
NPAD = 10240
RPS = NPAD // NS
ZR = 128
MB = 2000
G = N // MB

@functools.lru_cache(maxsize=None)
def _sc_params():
    cp = pltpu.CompilerParams()
    if "needs_layout_passes" in pltpu.CompilerParams.__dataclass_fields__:
        cp = dataclasses.replace(cp, needs_layout_passes=False)
    return cp


@functools.lru_cache(maxsize=None)
def _mesh():
    return plsc.VectorSubcoreMesh(
        core_axis_name="c", subcore_axis_name="s", num_cores=NC, num_subcores=NS
    )


def _sc_hist(ei3):
    @functools.partial(
        pl.kernel,
        out_type=(
            jax.ShapeDtypeStruct((NPAD,), jnp.float32),
            jax.ShapeDtypeStruct((NPAD,), jnp.float32),
        ),
        mesh=_mesh(),
        scratch_types=[
            pltpu.VMEM_SHARED((NS, NPAD), jnp.float32),
            pltpu.VMEM((EPS,), jnp.int32),
            pltpu.VMEM((NPAD,), jnp.float32),
            pltpu.VMEM((NS, RPS), jnp.float32),
            pltpu.VMEM((RPS,), jnp.float32),
        ],
        compiler_params=_sc_params(),
    )
    def k(ei_hbm, cr_hbm, cc_hbm, shared, idxv, local, mbuf, res):
        cid = lax.axis_index("c")
        sid = lax.axis_index("s")

        @pl.loop(0, NPAD, step=16)
        def _(i):
            local[pl.ds(i, 16)] = jnp.zeros((16,), jnp.float32)

        def run(which, out_hbm):
            pltpu.sync_copy(ei_hbm.at[which, sid], idxv)
            ones16 = jnp.ones((16,), jnp.float32)

            @pl.loop(0, EPS, step=16)
            def _(e):
                plsc.addupdate_scatter(local, [idxv[pl.ds(e, 16)]], ones16)

            pltpu.sync_copy(local, shared.at[sid])
            plsc.subcore_barrier()
            for r in range(NS):
                pltpu.sync_copy(shared.at[r, pl.ds(sid * RPS, RPS)],
                                mbuf.at[r])

            @pl.loop(0, RPS, step=16)
            def _(j):
                acc = mbuf[0, pl.ds(j, 16)]
                for r in range(1, NS):
                    acc = acc + mbuf[r, pl.ds(j, 16)]
                res[pl.ds(j, 16)] = acc

            pltpu.sync_copy(res, out_hbm.at[pl.ds(sid * RPS, RPS)])

        @pl.when(cid == 0)
        def _():
            run(0, cr_hbm)

        @pl.when(cid == 1)
        def _():
            run(1, cc_hbm)

    return k(ei3)


def _sc_propagate(ya, yb, ei4):
    @functools.partial(
        pl.kernel,
        out_type=(
            jax.ShapeDtypeStruct((NPAD, HALF), jnp.float32),
            jax.ShapeDtypeStruct((NPAD, HALF), jnp.float32),
        ),
        mesh=_mesh(),
        scratch_types=[
            pltpu.VMEM_SHARED((NPAD, HALF), jnp.float32),
            pltpu.VMEM((CH, B), jnp.int32),
            pltpu.VMEM((CH, B), jnp.int32),
            pltpu.VMEM((B, HALF), jnp.float32),
            pltpu.VMEM((B, HALF), jnp.float32),
            pltpu.SemaphoreType.DMA,
            pltpu.SemaphoreType.DMA,
        ],
    )
    def k(ya_hbm, yb_hbm, ei_hbm, za_hbm, zb_hbm,
          shared, rowc, colc, buf0, buf1, sem0, sem1):
        cid = lax.axis_index("c")
        sid = lax.axis_index("s")

        @pl.loop(0, B)
        def _(i):
            @pl.loop(0, HALF, step=16)
            def _(j):
                buf0[i, pl.ds(j, 16)] = jnp.zeros((16,), jnp.float32)

        @pl.loop(0, RPS // B)
        def _(i):
            pltpu.sync_copy(buf0, shared.at[pl.ds(sid * RPS + i * B, B)])

        plsc.subcore_barrier()

        def run(y_hbm, z_hbm):
            @pl.loop(0, NCH)
            def _(g):
                blk = sid * NCH + g
                pltpu.sync_copy(ei_hbm.at[0, blk], rowc)
                pltpu.sync_copy(ei_hbm.at[1, blk], colc)
                pltpu.async_copy(y_hbm.at[rowc.at[0]], buf0, sem0)

                @pl.loop(0, CH - 1, step=2)
                def _(b):
                    pltpu.async_copy(y_hbm.at[rowc.at[b + 1]], buf1, sem1)
                    pltpu.make_async_copy(y_hbm.at[rowc.at[b]], buf0,
                                          sem0).wait()
                    pltpu.sync_copy(buf0, shared.at[colc.at[b]], add=True)
                    pltpu.async_copy(y_hbm.at[rowc.at[b + 2]], buf0, sem0)
                    pltpu.make_async_copy(y_hbm.at[rowc.at[b + 1]], buf1,
                                          sem1).wait()
                    pltpu.sync_copy(buf1, shared.at[colc.at[b + 1]], add=True)

                pltpu.make_async_copy(y_hbm.at[rowc.at[CH - 1]], buf0,
                                      sem0).wait()
                pltpu.sync_copy(buf0, shared.at[colc.at[CH - 1]], add=True)

            plsc.subcore_barrier()

            @pl.loop(0, RPS // ZR)
            def _(i):
                sl = pl.ds(sid * RPS + i * ZR, ZR)
                pltpu.sync_copy(shared.at[sl], z_hbm.at[sl])

        @pl.when(cid == 0)
        def _():
            run(ya_hbm, za_hbm)

        @pl.when(cid == 1)
        def _():
            run(yb_hbm, zb_hbm)

    return k(ya, yb, ei4)


def _tc_prep(cnt_r, cnt_c, x):
    def body(cr, cc, xr, dis_o, dis2_o, ya_o, yb_o):
        dis = lax.rsqrt(cr[...] + 1.0)
        dis2 = lax.rsqrt(cc[...] + 1.0)
        dis_o[...] = dis
        dis2_o[...] = dis2
        ya_o[...] = dis * xr[:, :HALF]
        yb_o[...] = dis * xr[:, HALF:]

    return pl.pallas_call(
        body,
        grid=(G,),
        in_specs=[
            pl.BlockSpec((MB, 1), lambda i: (i, 0)),
            pl.BlockSpec((MB, 1), lambda i: (i, 0)),
            pl.BlockSpec((MB, DIN), lambda i: (i, 0)),
        ],
        out_specs=[
            pl.BlockSpec((MB, 1), lambda i: (i, 0)),
            pl.BlockSpec((MB, 1), lambda i: (i, 0)),
            pl.BlockSpec((MB, HALF), lambda i: (i, 0)),
            pl.BlockSpec((MB, HALF), lambda i: (i, 0)),
        ],
        out_shape=(
            jax.ShapeDtypeStruct((N, 1), jnp.float32),
            jax.ShapeDtypeStruct((N, 1), jnp.float32),
            jax.ShapeDtypeStruct((N, HALF), jnp.float32),
            jax.ShapeDtypeStruct((N, HALF), jnp.float32),
        ),
    )(cnt_r, cnt_c, x)


def _tc_mid(z1a, z1b, y1a, y1b, dis, dis2, W1, b1r, W2):
    def body(za, zb, ya, yb, d1, d2, w1, b1, w2, oa, ob):
        t = jnp.concatenate([za[...] + ya[...], zb[...] + yb[...]], axis=1)
        t = (t * d1[...]).astype(jnp.bfloat16)
        h = jnp.dot(t, w1[...].astype(jnp.bfloat16),
                    preferred_element_type=jnp.float32) + b1[...]
        h = jnp.maximum(h, 0.0).astype(jnp.bfloat16)
        y2 = jnp.dot(h, w2[...].astype(jnp.bfloat16),
                     preferred_element_type=jnp.float32) * d2[...]
        oa[...] = y2[:, :HALF]
        ob[...] = y2[:, HALF:]

    return pl.pallas_call(
        body,
        grid=(G,),
        in_specs=[
            pl.BlockSpec((MB, HALF), lambda i: (i, 0)),
            pl.BlockSpec((MB, HALF), lambda i: (i, 0)),
            pl.BlockSpec((MB, HALF), lambda i: (i, 0)),
            pl.BlockSpec((MB, HALF), lambda i: (i, 0)),
            pl.BlockSpec((MB, 1), lambda i: (i, 0)),
            pl.BlockSpec((MB, 1), lambda i: (i, 0)),
            pl.BlockSpec((DIN, DH), lambda i: (0, 0)),
            pl.BlockSpec((1, DH), lambda i: (0, 0)),
            pl.BlockSpec((DH, DOUT), lambda i: (0, 0)),
        ],
        out_specs=[
            pl.BlockSpec((MB, HALF), lambda i: (i, 0)),
            pl.BlockSpec((MB, HALF), lambda i: (i, 0)),
        ],
        out_shape=(
            jax.ShapeDtypeStruct((N, HALF), jnp.float32),
            jax.ShapeDtypeStruct((N, HALF), jnp.float32),
        ),
    )(z1a, z1b, y1a, y1b, dis, dis2, W1, b1r, W2)


def _tc_final(z2a, z2b, y2a, y2b, dis2, b2r):
    def body(za, zb, ya, yb, d2, b2, o):
        t = jnp.concatenate([za[...] + ya[...], zb[...] + yb[...]], axis=1)
        o[...] = t * d2[...] + b2[...]

    return pl.pallas_call(
        body,
        grid=(G,),
        in_specs=[
            pl.BlockSpec((MB, HALF), lambda i: (i, 0)),
            pl.BlockSpec((MB, HALF), lambda i: (i, 0)),
            pl.BlockSpec((MB, HALF), lambda i: (i, 0)),
            pl.BlockSpec((MB, HALF), lambda i: (i, 0)),
            pl.BlockSpec((MB, 1), lambda i: (i, 0)),
            pl.BlockSpec((1, DOUT), lambda i: (0, 0)),
        ],
        out_specs=pl.BlockSpec((MB, DOUT), lambda i: (i, 0)),
        out_shape=jax.ShapeDtypeStruct((N, DOUT), jnp.float32),
    )(z2a, z2b, y2a, y2b, dis2, b2r)


def kernel(x, edge_index, W1, b1, W2, b2):
    ei3 = edge_index.reshape(2, NS, EPS)
    ei4 = edge_index.reshape(2, NS * NCH, CH, B)
    cnt_r, cnt_c = _sc_hist(ei3)
    dis, dis2, y1a, y1b = _tc_prep(cnt_r.reshape(NPAD, 1),
                                   cnt_c.reshape(NPAD, 1), x)
    z1a, z1b = _sc_propagate(y1a, y1b, ei4)
    y2a, y2b = _tc_mid(z1a, z1b, y1a, y1b, dis, dis2, W1,
                       b1.reshape(1, DH), W2)
    z2a, z2b = _sc_propagate(y2a, y2b, ei4)
    return _tc_final(z2a, z2b, y2a, y2b, dis2, b2.reshape(1, DOUT))


# speedup vs baseline: 1.1357x; 1.0145x over previous
import dataclasses
import functools

import jax
import jax.numpy as jnp
from jax import lax
from jax.experimental import pallas as pl
from jax.experimental.pallas import tpu as pltpu
from jax.experimental.pallas import tpu_sc as plsc

N = 10000
E = 160000
DIN = 256
DH = 512
DOUT = 256
HALF = 128

NC = 2
NS = 16
B = 80
NB = E // NS // B
CH = 25
NCH = NB // CH
EPS = E // NS
NPAD = 10240
RPS = NPAD // NS
ZR = 128
MB = 2048
G = (N + MB - 1) // MB

@functools.lru_cache(maxsize=None)
def _sc_params():
    cp = pltpu.CompilerParams()
    if "needs_layout_passes" in pltpu.CompilerParams.__dataclass_fields__:
        cp = dataclasses.replace(cp, needs_layout_passes=False)
    return cp


@functools.lru_cache(maxsize=None)
def _mesh():
    return plsc.VectorSubcoreMesh(
        core_axis_name="c", subcore_axis_name="s", num_cores=NC, num_subcores=NS
    )


def _sc_hist(ei4):
    @functools.partial(
        pl.kernel,
        out_type=(
            jax.ShapeDtypeStruct((NPAD,), jnp.float32),
            jax.ShapeDtypeStruct((NPAD,), jnp.float32),
        ),
        mesh=_mesh(),
        scratch_types=[
            pltpu.VMEM_SHARED((NS, NPAD), jnp.float32),
            pltpu.VMEM((CH, B), jnp.int32),
            pltpu.VMEM((NPAD,), jnp.float32),
            pltpu.VMEM((NS, RPS), jnp.float32),
            pltpu.VMEM((RPS,), jnp.float32),
        ],
        compiler_params=_sc_params(),
    )
    def k(ei_hbm, cr_hbm, cc_hbm, shared, idxv, local, mbuf, res):
        cid = lax.axis_index("c")
        sid = lax.axis_index("s")

        @pl.loop(0, NPAD, step=16)
        def _(i):
            local[pl.ds(i, 16)] = jnp.zeros((16,), jnp.float32)

        def run(which, out_hbm):
            ones16 = jnp.ones((16,), jnp.float32)

            @pl.loop(0, NCH)
            def _(g):
                pltpu.sync_copy(ei_hbm.at[which, sid * NCH + g], idxv)

                @pl.loop(0, CH)
                def _(c):
                    @pl.loop(0, B, step=16)
                    def _(j):
                        plsc.addupdate_scatter(
                            local, [idxv[c, pl.ds(j, 16)]], ones16)

            pltpu.sync_copy(local, shared.at[sid])
            plsc.subcore_barrier()
            for r in range(NS):
                pltpu.sync_copy(shared.at[r, pl.ds(sid * RPS, RPS)],
                                mbuf.at[r])

            @pl.loop(0, RPS, step=16)
            def _(j):
                acc = mbuf[0, pl.ds(j, 16)]
                for r in range(1, NS):
                    acc = acc + mbuf[r, pl.ds(j, 16)]
                res[pl.ds(j, 16)] = acc

            pltpu.sync_copy(res, out_hbm.at[pl.ds(sid * RPS, RPS)])

        @pl.when(cid == 0)
        def _():
            run(0, cr_hbm)

        @pl.when(cid == 1)
        def _():
            run(1, cc_hbm)

    return k(ei4)


def _sc_propagate(ya, yb, ei4):
    @functools.partial(
        pl.kernel,
        out_type=(
            jax.ShapeDtypeStruct((NPAD, HALF), jnp.float32),
            jax.ShapeDtypeStruct((NPAD, HALF), jnp.float32),
        ),
        mesh=_mesh(),
        scratch_types=[
            pltpu.VMEM_SHARED((NPAD, HALF), jnp.float32),
            pltpu.VMEM((CH, B), jnp.int32),
            pltpu.VMEM((CH, B), jnp.int32),
            pltpu.VMEM((B, HALF), jnp.float32),
            pltpu.VMEM((B, HALF), jnp.float32),
            pltpu.SemaphoreType.DMA,
            pltpu.SemaphoreType.DMA,
        ],
    )
    def k(ya_hbm, yb_hbm, ei_hbm, za_hbm, zb_hbm,
          shared, rowc, colc, buf0, buf1, sem0, sem1):
        cid = lax.axis_index("c")
        sid = lax.axis_index("s")

        @pl.loop(0, B)
        def _(i):
            @pl.loop(0, HALF, step=16)
            def _(j):
                buf0[i, pl.ds(j, 16)] = jnp.zeros((16,), jnp.float32)

        @pl.loop(0, RPS // B)
        def _(i):
            pltpu.sync_copy(buf0, shared.at[pl.ds(sid * RPS + i * B, B)])

        plsc.subcore_barrier()

        def run(y_hbm, z_hbm):
            @pl.loop(0, NCH)
            def _(g):
                blk = sid * NCH + g
                pltpu.sync_copy(ei_hbm.at[0, blk], rowc)
                pltpu.sync_copy(ei_hbm.at[1, blk], colc)
                pltpu.async_copy(y_hbm.at[rowc.at[0]], buf0, sem0)

                @pl.loop(0, CH - 1, step=2)
                def _(b):
                    pltpu.async_copy(y_hbm.at[rowc.at[b + 1]], buf1, sem1)
                    pltpu.make_async_copy(y_hbm.at[rowc.at[b]], buf0,
                                          sem0).wait()
                    pltpu.sync_copy(buf0, shared.at[colc.at[b]], add=True)
                    pltpu.async_copy(y_hbm.at[rowc.at[b + 2]], buf0, sem0)
                    pltpu.make_async_copy(y_hbm.at[rowc.at[b + 1]], buf1,
                                          sem1).wait()
                    pltpu.sync_copy(buf1, shared.at[colc.at[b + 1]], add=True)

                pltpu.make_async_copy(y_hbm.at[rowc.at[CH - 1]], buf0,
                                      sem0).wait()
                pltpu.sync_copy(buf0, shared.at[colc.at[CH - 1]], add=True)

            plsc.subcore_barrier()

            @pl.loop(0, RPS // ZR)
            def _(i):
                sl = pl.ds(sid * RPS + i * ZR, ZR)
                pltpu.sync_copy(shared.at[sl], z_hbm.at[sl])

        @pl.when(cid == 0)
        def _():
            run(ya_hbm, za_hbm)

        @pl.when(cid == 1)
        def _():
            run(yb_hbm, zb_hbm)

    return k(ya, yb, ei4)


def _tc_prep(cnt_r, cnt_c, x):
    def body(cr, cc, xr, dis_o, dis2_o, ya_o, yb_o):
        i = pl.program_id(0)
        dis = lax.rsqrt(cr[pl.ds(i * MB, MB)].reshape(MB, 1) + 1.0)
        dis2 = lax.rsqrt(cc[pl.ds(i * MB, MB)].reshape(MB, 1) + 1.0)
        dis_o[...] = dis
        dis2_o[...] = dis2
        ya_o[...] = dis * xr[:, :HALF]
        yb_o[...] = dis * xr[:, HALF:]

    return pl.pallas_call(
        body,
        grid=(G,),
        in_specs=[
            pl.BlockSpec((NPAD,), lambda i: (0,)),
            pl.BlockSpec((NPAD,), lambda i: (0,)),
            pl.BlockSpec((MB, DIN), lambda i: (i, 0)),
        ],
        out_specs=[
            pl.BlockSpec((MB, 1), lambda i: (i, 0)),
            pl.BlockSpec((MB, 1), lambda i: (i, 0)),
            pl.BlockSpec((MB, HALF), lambda i: (i, 0)),
            pl.BlockSpec((MB, HALF), lambda i: (i, 0)),
        ],
        out_shape=(
            jax.ShapeDtypeStruct((N, 1), jnp.float32),
            jax.ShapeDtypeStruct((N, 1), jnp.float32),
            jax.ShapeDtypeStruct((N, HALF), jnp.float32),
            jax.ShapeDtypeStruct((N, HALF), jnp.float32),
        ),
    )(cnt_r, cnt_c, x)


def _tc_mid(z1a, z1b, y1a, y1b, dis, dis2, W1, b1r, W2):
    def body(za, zb, ya, yb, d1, d2, w1, b1, w2, oa, ob):
        t = jnp.concatenate([za[...] + ya[...], zb[...] + yb[...]], axis=1)
        t = (t * d1[...]).astype(jnp.bfloat16)
        h = jnp.dot(t, w1[...].astype(jnp.bfloat16),
                    preferred_element_type=jnp.float32) + b1[...]
        h = jnp.maximum(h, 0.0).astype(jnp.bfloat16)
        y2 = jnp.dot(h, w2[...].astype(jnp.bfloat16),
                     preferred_element_type=jnp.float32) * d2[...]
        oa[...] = y2[:, :HALF]
        ob[...] = y2[:, HALF:]

    return pl.pallas_call(
        body,
        grid=(G,),
        in_specs=[
            pl.BlockSpec((MB, HALF), lambda i: (i, 0)),
            pl.BlockSpec((MB, HALF), lambda i: (i, 0)),
            pl.BlockSpec((MB, HALF), lambda i: (i, 0)),
            pl.BlockSpec((MB, HALF), lambda i: (i, 0)),
            pl.BlockSpec((MB, 1), lambda i: (i, 0)),
            pl.BlockSpec((MB, 1), lambda i: (i, 0)),
            pl.BlockSpec((DIN, DH), lambda i: (0, 0)),
            pl.BlockSpec((1, DH), lambda i: (0, 0)),
            pl.BlockSpec((DH, DOUT), lambda i: (0, 0)),
        ],
        out_specs=[
            pl.BlockSpec((MB, HALF), lambda i: (i, 0)),
            pl.BlockSpec((MB, HALF), lambda i: (i, 0)),
        ],
        out_shape=(
            jax.ShapeDtypeStruct((N, HALF), jnp.float32),
            jax.ShapeDtypeStruct((N, HALF), jnp.float32),
        ),
    )(z1a, z1b, y1a, y1b, dis, dis2, W1, b1r, W2)


def _tc_final(z2a, z2b, y2a, y2b, dis2, b2r):
    def body(za, zb, ya, yb, d2, b2, o):
        t = jnp.concatenate([za[...] + ya[...], zb[...] + yb[...]], axis=1)
        o[...] = t * d2[...] + b2[...]

    return pl.pallas_call(
        body,
        grid=(G,),
        in_specs=[
            pl.BlockSpec((MB, HALF), lambda i: (i, 0)),
            pl.BlockSpec((MB, HALF), lambda i: (i, 0)),
            pl.BlockSpec((MB, HALF), lambda i: (i, 0)),
            pl.BlockSpec((MB, HALF), lambda i: (i, 0)),
            pl.BlockSpec((MB, 1), lambda i: (i, 0)),
            pl.BlockSpec((1, DOUT), lambda i: (0, 0)),
        ],
        out_specs=pl.BlockSpec((MB, DOUT), lambda i: (i, 0)),
        out_shape=jax.ShapeDtypeStruct((N, DOUT), jnp.float32),
    )(z2a, z2b, y2a, y2b, dis2, b2r)


def kernel(x, edge_index, W1, b1, W2, b2):
    ei4 = edge_index.reshape(2, NS * NCH, CH, B)
    cnt_r, cnt_c = _sc_hist(ei4)
    dis, dis2, y1a, y1b = _tc_prep(cnt_r, cnt_c, x)
    z1a, z1b = _sc_propagate(y1a, y1b, ei4)
    y2a, y2b = _tc_mid(z1a, z1b, y1a, y1b, dis, dis2, W1,
                       b1.reshape(1, DH), W2)
    z2a, z2b = _sc_propagate(y2a, y2b, ei4)
    return _tc_final(z2a, z2b, y2a, y2b, dis2, b2.reshape(1, DOUT))
